# baseline (speedup 1.0000x reference)
import functools
import math

import numpy as np
import jax
import jax.numpy as jnp
from jax import lax
from jax.experimental import pallas as pl
from jax.experimental.pallas import tpu as pltpu
from jax.experimental.pallas import tpu_sc as plsc

N = 10000
E = 320000
D = 128
NUM_ELEM = 10
NUM_BESSEL = 8
HIDDEN = 64
AVG_NEIGH = 32.0
_SILU_NORM = 1.6790532

NC = 2
NS = 16
L = 16
NW = NC * NS

K = 80
CHUNKS = E // K
CPW = CHUNKS // NW
NP = 10240
RPT = NP // NS
_MASKHI = -65536

DW = D // 2


def _silu(x):
    return x * jax.nn.sigmoid(x) * _SILU_NORM


def _pack_rows(x):
    bits = lax.bitcast_convert_type(x, jnp.int32) + 0x8000
    words = []
    for j in range(4):
        a = bits[:, 32 * j:32 * j + 16]
        b = bits[:, 32 * j + 16:32 * j + 32]
        words.append(jnp.bitwise_or(lax.shift_right_logical(a, 16),
                                    jnp.bitwise_and(b, _MASKHI)))
    return jnp.concatenate(words, axis=1)


def _h_body(nf_ref, wup_ref, h_ref):
    h_ref[...] = jnp.dot(nf_ref[...], wup_ref[...],
                         preferred_element_type=jnp.float32) * (1.0 / math.sqrt(D))


def _node_linear(node_feats, W_up):
    BN = 2000
    return pl.pallas_call(
        _h_body,
        out_shape=jax.ShapeDtypeStruct((N, D), jnp.float32),
        grid=(N // BN,),
        in_specs=[pl.BlockSpec((BN, D), lambda i: (i, 0)),
                  pl.BlockSpec((D, D), lambda i: (0, 0))],
        out_specs=pl.BlockSpec((BN, D), lambda i: (i, 0)),
    )(node_feats, W_up)


def _coeff_body(ef_ref, ea_ref, w0_ref, w1_ref, w2_ref, w3_ref, out_ref):
    x = jnp.dot(ef_ref[...], w0_ref[...],
                preferred_element_type=jnp.float32) * (1.0 / math.sqrt(NUM_BESSEL))
    x = _silu(x)
    x = jnp.dot(x, w1_ref[...],
                preferred_element_type=jnp.float32) * (1.0 / math.sqrt(HIDDEN))
    x = _silu(x)
    x = jnp.dot(x, w2_ref[...],
                preferred_element_type=jnp.float32) * (1.0 / math.sqrt(HIDDEN))
    x = _silu(x)
    tw = jnp.dot(x, w3_ref[...],
                 preferred_element_type=jnp.float32) * (1.0 / math.sqrt(HIDDEN))
    out_ref[...] = _pack_rows(tw * ea_ref[...])


def _edge_coeff(edge_feats, edge_attrs, W_r0, W_r1, W_r2, W_r3p):
    BE = 2000
    return pl.pallas_call(
        _coeff_body,
        out_shape=jax.ShapeDtypeStruct((E, DW), jnp.int32),
        grid=(E // BE,),
        in_specs=[pl.BlockSpec((BE, NUM_BESSEL), lambda i: (i, 0)),
                  pl.BlockSpec((BE, 1), lambda i: (i, 0)),
                  pl.BlockSpec((NUM_BESSEL, HIDDEN), lambda i: (0, 0)),
                  pl.BlockSpec((HIDDEN, HIDDEN), lambda i: (0, 0)),
                  pl.BlockSpec((HIDDEN, HIDDEN), lambda i: (0, 0)),
                  pl.BlockSpec((HIDDEN, D), lambda i: (0, 0))],
        out_specs=pl.BlockSpec((BE, DW), lambda i: (i, 0)),
    )(edge_feats, edge_attrs, W_r0, W_r1, W_r2, W_r3p)


def _sc_body(h_hbm, coeff_hbm, src_hbm, dst_hbm, out_hbm,
             src0_v, src1_v, dst0_v, dst1_v, hr0_v, hr1_v,
             cf0_v, cf1_v, msg_sh,
             sg0, sg1, sc0, sc1, ss0, ss1):
    c = lax.axis_index("c")
    s = lax.axis_index("s")
    wid = s * NC + c
    bufs = ((src0_v, dst0_v, hr0_v, cf0_v, sg0, sc0, ss0),
            (src1_v, dst1_v, hr1_v, cf1_v, sg1, sc1, ss1))

    zero = jnp.zeros((L,), jnp.float32)

    def zrow(r, carry):
        for j in range(D // L):
            hr0_v[r, pl.ds(j * L, L)] = zero
        return carry

    lax.fori_loop(0, K, zrow, 0)
    base = s * RPT
    for t in range(RPT // K):
        pltpu.sync_copy(hr0_v, msg_sh.at[pl.ds(base + t * K, K)])
    plsc.subcore_barrier()

    def _mult(hr, cf):
        def mrow(r):
            for t in range(4):
                cv = cf[r, pl.ds(16 * t, 16)]
                ac = lax.bitcast_convert_type(lax.shift_left(cv, 16),
                                              jnp.float32)
                bc = lax.bitcast_convert_type(jnp.bitwise_and(cv, _MASKHI),
                                              jnp.float32)
                slo = pl.ds(32 * t, 16)
                shi = pl.ds(32 * t + 16, 16)
                hr[r, slo] = hr[r, slo] * ac
                hr[r, shi] = hr[r, shi] * bc

        plsc.parallel_loop(0, K, 1, unroll=2)(mrow)

    def body(g, carry):
        for b in (0, 1):
            src_v, dst_v, hr, cf, sg, sc_, ss = bufs[b]

            @pl.when(g >= 1)
            def _():
                pltpu.make_async_copy(hr, msg_sh.at[dst_v.at[0]], ss).wait()

            chunk = wid + (2 * g + b) * NW
            pltpu.sync_copy(src_hbm.at[pl.ds(chunk * K, K)], src_v)
            pltpu.sync_copy(dst_hbm.at[pl.ds(chunk * K, K)], dst_v.at[0])
            pltpu.async_copy(h_hbm.at[src_v], hr, sg)
            pltpu.async_copy(coeff_hbm.at[pl.ds(chunk * K, K)], cf, sc_)
        for b in (0, 1):
            src_v, dst_v, hr, cf, sg, sc_, ss = bufs[b]
            chunk = wid + (2 * g + b) * NW
            pltpu.make_async_copy(h_hbm.at[src_v], hr, sg).wait()
            pltpu.make_async_copy(coeff_hbm.at[pl.ds(chunk * K, K)], cf,
                                  sc_).wait()
            _mult(hr, cf)
            pltpu.async_copy(hr, msg_sh.at[dst_v.at[0]], ss, add=True)
        return carry

    lax.fori_loop(0, CPW // 2, body, 0)
    for b in (0, 1):
        src_v, dst_v, hr, cf, sg, sc_, ss = bufs[b]
        pltpu.make_async_copy(hr, msg_sh.at[dst_v.at[0]], ss).wait()

    tchunk = wid + (CPW - 1) * NW
    pltpu.sync_copy(src_hbm.at[pl.ds(tchunk * K, K)], src0_v)
    pltpu.sync_copy(dst_hbm.at[pl.ds(tchunk * K, K)], dst0_v.at[0])
    pltpu.async_copy(h_hbm.at[src0_v], hr0_v, sg0).wait()
    pltpu.async_copy(coeff_hbm.at[pl.ds(tchunk * K, K)], cf0_v, sc0).wait()
    _mult(hr0_v, cf0_v)
    pltpu.async_copy(hr0_v, msg_sh.at[dst0_v.at[0]], ss0, add=True).wait()

    plsc.subcore_barrier()
    pltpu.sync_copy(msg_sh.at[pl.ds(base, RPT)], out_hbm.at[c, pl.ds(base, RPT)])


def _sc_scatter(h, coeff_p, src_p, dst_p):
    mesh = plsc.VectorSubcoreMesh(core_axis_name="c", subcore_axis_name="s",
                                  num_cores=NC, num_subcores=NS)
    fn = pl.kernel(
        _sc_body,
        out_type=jax.ShapeDtypeStruct((NC, NP, D), jnp.float32),
        mesh=mesh,
        scratch_types=[
            pltpu.VMEM((K,), jnp.int32),
            pltpu.VMEM((K,), jnp.int32),
            pltpu.VMEM((1, K), jnp.int32),
            pltpu.VMEM((1, K), jnp.int32),
            pltpu.VMEM((K, D), jnp.float32),
            pltpu.VMEM((K, D), jnp.float32),
            pltpu.VMEM((K, DW), jnp.int32),
            pltpu.VMEM((K, DW), jnp.int32),
            pltpu.VMEM_SHARED((NP, D), jnp.float32),
            pltpu.SemaphoreType.DMA,
            pltpu.SemaphoreType.DMA,
            pltpu.SemaphoreType.DMA,
            pltpu.SemaphoreType.DMA,
            pltpu.SemaphoreType.DMA,
            pltpu.SemaphoreType.DMA,
        ],
    )
    return fn(h, coeff_p, src_p, dst_p)


def _out_body(part_ref, attrs_ref, wlin_ref, wskt_ref, out_ref):
    m = part_ref[0] + part_ref[1]
    m2 = jnp.dot(m, wlin_ref[...], preferred_element_type=jnp.float32) * (
        1.0 / (math.sqrt(D) * AVG_NEIGH))
    attrs = attrs_ref[...]
    acc = attrs[:, 0][:, None] * jnp.dot(m2, wskt_ref[0],
                                         preferred_element_type=jnp.float32)
    for j in range(1, NUM_ELEM):
        acc = acc + attrs[:, j][:, None] * jnp.dot(
            m2, wskt_ref[j], preferred_element_type=jnp.float32)
    out_ref[...] = acc * (1.0 / math.sqrt(D * NUM_ELEM))


def _final(partials, node_attrs, W_lin, W_skip_t):
    BN = 2000
    return pl.pallas_call(
        _out_body,
        out_shape=jax.ShapeDtypeStruct((N, D), jnp.float32),
        grid=(N // BN,),
        in_specs=[pl.BlockSpec((NC, BN, D), lambda i: (0, i, 0)),
                  pl.BlockSpec((BN, NUM_ELEM), lambda i: (i, 0)),
                  pl.BlockSpec((D, D), lambda i: (0, 0)),
                  pl.BlockSpec((NUM_ELEM, D, D), lambda i: (0, 0, 0))],
        out_specs=pl.BlockSpec((BN, D), lambda i: (i, 0)),
    )(partials, node_attrs, W_lin, W_skip_t)


def kernel(node_feats, node_attrs, edge_feats, edge_attrs, edge_index,
           W_up, W_r0, W_r1, W_r2, W_r3, W_lin, W_skip):
    src_p = edge_index[0]
    dst_p = edge_index[1]

    h = _node_linear(node_feats, W_up)
    coeff = _edge_coeff(edge_feats, edge_attrs, W_r0, W_r1, W_r2, W_r3)
    partials = _sc_scatter(h, coeff, src_p, dst_p)
    return _final(partials, node_attrs, W_lin, W_skip.transpose(1, 0, 2))


# speedup vs baseline: 1.0731x; 1.0731x over previous
import functools
import math

import numpy as np
import jax
import jax.numpy as jnp
from jax import lax
from jax.experimental import pallas as pl
from jax.experimental.pallas import tpu as pltpu
from jax.experimental.pallas import tpu_sc as plsc

N = 10000
E = 320000
D = 128
NUM_ELEM = 10
NUM_BESSEL = 8
HIDDEN = 64
AVG_NEIGH = 32.0
_SILU_NORM = 1.6790532

NC = 2
NS = 16
L = 16
NW = NC * NS

K = 80
CHUNKS = E // K
CPW = CHUNKS // NW
NP = 10240
RPT = NP // NS
_MASKHI = -65536

DW = D // 2


def _silu(x):
    return x * jax.nn.sigmoid(x) * _SILU_NORM


def _pack_rows(x):
    bits = lax.bitcast_convert_type(x, jnp.int32) + 0x8000
    return jnp.bitwise_or(lax.shift_right_logical(bits[:, :DW], 16),
                          jnp.bitwise_and(bits[:, DW:], _MASKHI))


def _split_body(ei_ref, src_ref, dst_ref):
    ei = ei_ref[...]
    src_ref[...] = ei[0]
    dst_ref[...] = ei[1]


def _split_idx(edge_index):
    return pl.pallas_call(
        _split_body,
        out_shape=(jax.ShapeDtypeStruct((E,), jnp.int32),
                   jax.ShapeDtypeStruct((E,), jnp.int32)),
    )(edge_index)


def _h_body(nf_ref, wup_ref, h_ref):
    h_ref[...] = jnp.dot(nf_ref[...], wup_ref[...],
                         preferred_element_type=jnp.float32) * (1.0 / math.sqrt(D))


def _node_linear(node_feats, W_up):
    BN = 2000
    return pl.pallas_call(
        _h_body,
        out_shape=jax.ShapeDtypeStruct((N, D), jnp.float32),
        grid=(N // BN,),
        in_specs=[pl.BlockSpec((BN, D), lambda i: (i, 0)),
                  pl.BlockSpec((D, D), lambda i: (0, 0))],
        out_specs=pl.BlockSpec((BN, D), lambda i: (i, 0)),
    )(node_feats, W_up)


def _coeff_body(ef_ref, ea_ref, w0_ref, w1_ref, w2_ref, w3_ref, out_ref):
    x = jnp.dot(ef_ref[...], w0_ref[...],
                preferred_element_type=jnp.float32) * (1.0 / math.sqrt(NUM_BESSEL))
    x = _silu(x)
    x = jnp.dot(x, w1_ref[...],
                preferred_element_type=jnp.float32) * (1.0 / math.sqrt(HIDDEN))
    x = _silu(x)
    x = jnp.dot(x, w2_ref[...],
                preferred_element_type=jnp.float32) * (1.0 / math.sqrt(HIDDEN))
    x = _silu(x)
    tw = jnp.dot(x, w3_ref[...],
                 preferred_element_type=jnp.float32) * (1.0 / math.sqrt(HIDDEN))
    out_ref[...] = _pack_rows(tw * ea_ref[...])


def _edge_coeff(edge_feats, edge_attrs, W_r0, W_r1, W_r2, W_r3p):
    BE = 2000
    return pl.pallas_call(
        _coeff_body,
        out_shape=jax.ShapeDtypeStruct((E, DW), jnp.int32),
        grid=(E // BE,),
        in_specs=[pl.BlockSpec((BE, NUM_BESSEL), lambda i: (i, 0)),
                  pl.BlockSpec((BE, 1), lambda i: (i, 0)),
                  pl.BlockSpec((NUM_BESSEL, HIDDEN), lambda i: (0, 0)),
                  pl.BlockSpec((HIDDEN, HIDDEN), lambda i: (0, 0)),
                  pl.BlockSpec((HIDDEN, HIDDEN), lambda i: (0, 0)),
                  pl.BlockSpec((HIDDEN, D), lambda i: (0, 0))],
        out_specs=pl.BlockSpec((BE, DW), lambda i: (i, 0)),
    )(edge_feats, edge_attrs, W_r0, W_r1, W_r2, W_r3p)


def _sc_body(h_hbm, coeff_hbm, src_hbm, dst_hbm, out_hbm,
             src0_v, src1_v, dst0_v, dst1_v, hr0_v, hr1_v,
             cf0_v, cf1_v, msg_sh,
             sg0, sg1, sc0, sc1, ss0, ss1):
    c = lax.axis_index("c")
    s = lax.axis_index("s")
    wid = s * NC + c
    bufs = ((src0_v, dst0_v, hr0_v, cf0_v, sg0, sc0, ss0),
            (src1_v, dst1_v, hr1_v, cf1_v, sg1, sc1, ss1))

    zero = jnp.zeros((L,), jnp.float32)

    def zrow(r, carry):
        for j in range(D // L):
            hr0_v[r, pl.ds(j * L, L)] = zero
        return carry

    lax.fori_loop(0, K, zrow, 0)
    base = s * RPT
    for t in range(RPT // K):
        pltpu.sync_copy(hr0_v, msg_sh.at[pl.ds(base + t * K, K)])
    plsc.subcore_barrier()

    def _mult(hr, cf):
        def mrow(r):
            for t in range(4):
                cv = cf[r, pl.ds(16 * t, 16)]
                ac = lax.bitcast_convert_type(lax.shift_left(cv, 16),
                                              jnp.float32)
                bc = lax.bitcast_convert_type(jnp.bitwise_and(cv, _MASKHI),
                                              jnp.float32)
                slo = pl.ds(16 * t, 16)
                shi = pl.ds(DW + 16 * t, 16)
                hr[r, slo] = hr[r, slo] * ac
                hr[r, shi] = hr[r, shi] * bc

        plsc.parallel_loop(0, K, 1, unroll=2)(mrow)

    def body(g, carry):
        for b in (0, 1):
            src_v, dst_v, hr, cf, sg, sc_, ss = bufs[b]

            @pl.when(g >= 1)
            def _():
                pltpu.make_async_copy(hr, msg_sh.at[dst_v.at[0]], ss).wait()

            chunk = wid + (2 * g + b) * NW
            pltpu.sync_copy(src_hbm.at[pl.ds(chunk * K, K)], src_v)
            pltpu.sync_copy(dst_hbm.at[pl.ds(chunk * K, K)], dst_v.at[0])
            pltpu.async_copy(h_hbm.at[src_v], hr, sg)
            pltpu.async_copy(coeff_hbm.at[pl.ds(chunk * K, K)], cf, sc_)
        for b in (0, 1):
            src_v, dst_v, hr, cf, sg, sc_, ss = bufs[b]
            chunk = wid + (2 * g + b) * NW
            pltpu.make_async_copy(h_hbm.at[src_v], hr, sg).wait()
            pltpu.make_async_copy(coeff_hbm.at[pl.ds(chunk * K, K)], cf,
                                  sc_).wait()
            _mult(hr, cf)
            pltpu.async_copy(hr, msg_sh.at[dst_v.at[0]], ss, add=True)
        return carry

    lax.fori_loop(0, CPW // 2, body, 0)
    for b in (0, 1):
        src_v, dst_v, hr, cf, sg, sc_, ss = bufs[b]
        pltpu.make_async_copy(hr, msg_sh.at[dst_v.at[0]], ss).wait()

    tchunk = wid + (CPW - 1) * NW
    pltpu.sync_copy(src_hbm.at[pl.ds(tchunk * K, K)], src0_v)
    pltpu.sync_copy(dst_hbm.at[pl.ds(tchunk * K, K)], dst0_v.at[0])
    pltpu.async_copy(h_hbm.at[src0_v], hr0_v, sg0).wait()
    pltpu.async_copy(coeff_hbm.at[pl.ds(tchunk * K, K)], cf0_v, sc0).wait()
    _mult(hr0_v, cf0_v)
    pltpu.async_copy(hr0_v, msg_sh.at[dst0_v.at[0]], ss0, add=True).wait()

    plsc.subcore_barrier()
    pltpu.sync_copy(msg_sh.at[pl.ds(base, RPT)], out_hbm.at[c, pl.ds(base, RPT)])


def _sc_scatter(h, coeff_p, src_p, dst_p):
    mesh = plsc.VectorSubcoreMesh(core_axis_name="c", subcore_axis_name="s",
                                  num_cores=NC, num_subcores=NS)
    fn = pl.kernel(
        _sc_body,
        out_type=jax.ShapeDtypeStruct((NC, NP, D), jnp.float32),
        mesh=mesh,
        scratch_types=[
            pltpu.VMEM((K,), jnp.int32),
            pltpu.VMEM((K,), jnp.int32),
            pltpu.VMEM((1, K), jnp.int32),
            pltpu.VMEM((1, K), jnp.int32),
            pltpu.VMEM((K, D), jnp.float32),
            pltpu.VMEM((K, D), jnp.float32),
            pltpu.VMEM((K, DW), jnp.int32),
            pltpu.VMEM((K, DW), jnp.int32),
            pltpu.VMEM_SHARED((NP, D), jnp.float32),
            pltpu.SemaphoreType.DMA,
            pltpu.SemaphoreType.DMA,
            pltpu.SemaphoreType.DMA,
            pltpu.SemaphoreType.DMA,
            pltpu.SemaphoreType.DMA,
            pltpu.SemaphoreType.DMA,
        ],
    )
    return fn(h, coeff_p, src_p, dst_p)


def _out_body(part_ref, attrs_ref, wlin_ref, wskt_ref, out_ref):
    m = part_ref[0] + part_ref[1]
    m2 = jnp.dot(m, wlin_ref[...], preferred_element_type=jnp.float32) * (
        1.0 / (math.sqrt(D) * AVG_NEIGH))
    attrs = attrs_ref[...]
    acc = attrs[:, 0][:, None] * jnp.dot(m2, wskt_ref[0],
                                         preferred_element_type=jnp.float32)
    for j in range(1, NUM_ELEM):
        acc = acc + attrs[:, j][:, None] * jnp.dot(
            m2, wskt_ref[j], preferred_element_type=jnp.float32)
    out_ref[...] = acc * (1.0 / math.sqrt(D * NUM_ELEM))


def _final(partials, node_attrs, W_lin, W_skip_t):
    BN = 2000
    return pl.pallas_call(
        _out_body,
        out_shape=jax.ShapeDtypeStruct((N, D), jnp.float32),
        grid=(N // BN,),
        in_specs=[pl.BlockSpec((NC, BN, D), lambda i: (0, i, 0)),
                  pl.BlockSpec((BN, NUM_ELEM), lambda i: (i, 0)),
                  pl.BlockSpec((D, D), lambda i: (0, 0)),
                  pl.BlockSpec((NUM_ELEM, D, D), lambda i: (0, 0, 0))],
        out_specs=pl.BlockSpec((BN, D), lambda i: (i, 0)),
    )(partials, node_attrs, W_lin, W_skip_t)


def kernel(node_feats, node_attrs, edge_feats, edge_attrs, edge_index,
           W_up, W_r0, W_r1, W_r2, W_r3, W_lin, W_skip):
    src_p, dst_p = _split_idx(edge_index)
    h = _node_linear(node_feats, W_up)
    coeff = _edge_coeff(edge_feats, edge_attrs, W_r0, W_r1, W_r2, W_r3)
    partials = _sc_scatter(h, coeff, src_p, dst_p)
    return _final(partials, node_attrs, W_lin, W_skip.transpose(1, 0, 2))


# speedup vs baseline: 1.1857x; 1.1049x over previous
import functools
import math

import numpy as np
import jax
import jax.numpy as jnp
from jax import lax
from jax.experimental import pallas as pl
from jax.experimental.pallas import tpu as pltpu
from jax.experimental.pallas import tpu_sc as plsc

N = 10000
E = 320000
D = 128
NUM_ELEM = 10
NUM_BESSEL = 8
HIDDEN = 64
AVG_NEIGH = 32.0
_SILU_NORM = 1.6790532

NC = 2
NS = 16
L = 16
NW = NC * NS

K = 80
CHUNKS = E // K
CPW = CHUNKS // NW
NP = 10240
RPT = NP // NS
_MASKHI = -65536

DW = D // 2


def _silu(x):
    return x * jax.nn.sigmoid(x) * _SILU_NORM


def _pack_rows(x):
    bits = lax.bitcast_convert_type(x, jnp.int32) + 0x8000
    return jnp.bitwise_or(lax.shift_right_logical(bits[:, :DW], 16),
                          jnp.bitwise_and(bits[:, DW:], _MASKHI))


def _split_body(ei_ref, src_ref, dst_ref):
    ei = ei_ref[...]
    src_ref[...] = ei[0]
    dst_ref[...] = ei[1]


def _split_idx(edge_index):
    return pl.pallas_call(
        _split_body,
        out_shape=(jax.ShapeDtypeStruct((E,), jnp.int32),
                   jax.ShapeDtypeStruct((E,), jnp.int32)),
    )(edge_index)


def _h_body(nf_ref, wup_ref, h_ref):
    h_ref[...] = jnp.dot(nf_ref[...], wup_ref[...],
                         preferred_element_type=jnp.float32) * (1.0 / math.sqrt(D))


def _node_linear(node_feats, W_up):
    BN = 2000
    return pl.pallas_call(
        _h_body,
        out_shape=jax.ShapeDtypeStruct((N, D), jnp.float32),
        grid=(N // BN,),
        in_specs=[pl.BlockSpec((BN, D), lambda i: (i, 0)),
                  pl.BlockSpec((D, D), lambda i: (0, 0))],
        out_specs=pl.BlockSpec((BN, D), lambda i: (i, 0)),
    )(node_feats, W_up)


def _coeff_body(ef_ref, ea_ref, w0_ref, w1_ref, w2_ref, w3_ref, out_ref):
    x = jnp.dot(ef_ref[...], w0_ref[...],
                preferred_element_type=jnp.float32) * (1.0 / math.sqrt(NUM_BESSEL))
    x = _silu(x)
    x = jnp.dot(x, w1_ref[...],
                preferred_element_type=jnp.float32) * (1.0 / math.sqrt(HIDDEN))
    x = _silu(x)
    x = jnp.dot(x, w2_ref[...],
                preferred_element_type=jnp.float32) * (1.0 / math.sqrt(HIDDEN))
    x = _silu(x)
    tw = jnp.dot(x, w3_ref[...],
                 preferred_element_type=jnp.float32) * (1.0 / math.sqrt(HIDDEN))
    out_ref[...] = _pack_rows(tw * ea_ref[...])


def _edge_coeff(edge_feats, edge_attrs, W_r0, W_r1, W_r2, W_r3p):
    BE = 2000
    return pl.pallas_call(
        _coeff_body,
        out_shape=jax.ShapeDtypeStruct((E, DW), jnp.int32),
        grid=(E // BE,),
        in_specs=[pl.BlockSpec((BE, NUM_BESSEL), lambda i: (i, 0)),
                  pl.BlockSpec((BE, 1), lambda i: (i, 0)),
                  pl.BlockSpec((NUM_BESSEL, HIDDEN), lambda i: (0, 0)),
                  pl.BlockSpec((HIDDEN, HIDDEN), lambda i: (0, 0)),
                  pl.BlockSpec((HIDDEN, HIDDEN), lambda i: (0, 0)),
                  pl.BlockSpec((HIDDEN, D), lambda i: (0, 0))],
        out_specs=pl.BlockSpec((BE, DW), lambda i: (i, 0)),
    )(edge_feats, edge_attrs, W_r0, W_r1, W_r2, W_r3p)


def _sc_body(h_hbm, coeff_hbm, src_hbm, dst_hbm, out_hbm,
             src0_v, src1_v, dst0_v, dst1_v, hr0_v, hr1_v,
             cf0_v, cf1_v, msg_sh,
             sg0, sg1, sc0, sc1, ss0, ss1, si0, si1):
    c = lax.axis_index("c")
    s = lax.axis_index("s")
    wid = s * NC + c
    bufs = ((src0_v, dst0_v, hr0_v, cf0_v, sg0, sc0, ss0, si0),
            (src1_v, dst1_v, hr1_v, cf1_v, sg1, sc1, ss1, si1))

    def issue_idx(b, i, p):
        src_v, dst_v, _, _, _, _, _, si = bufs[b]
        chunk = wid + i * NW
        pltpu.async_copy(src_hbm.at[pl.ds(chunk * K, K)], src_v, si)
        pltpu.async_copy(dst_hbm.at[pl.ds(chunk * K, K)], dst_v.at[p], si)

    def wait_idx(b, p):
        src_v, dst_v, _, _, _, _, _, si = bufs[b]
        pltpu.make_async_copy(src_hbm.at[pl.ds(0, K)], src_v, si).wait()
        pltpu.make_async_copy(dst_hbm.at[pl.ds(0, K)], dst_v.at[p], si).wait()

    zero = jnp.zeros((L,), jnp.float32)

    def zrow(r, carry):
        for j in range(D // L):
            hr0_v[r, pl.ds(j * L, L)] = zero
        return carry

    lax.fori_loop(0, K, zrow, 0)
    base = s * RPT
    for t in range(RPT // K):
        pltpu.sync_copy(hr0_v, msg_sh.at[pl.ds(base + t * K, K)])
    plsc.subcore_barrier()

    def _mult(hr, cf):
        def mrow(r):
            for t in range(4):
                cv = cf[r, pl.ds(16 * t, 16)]
                ac = lax.bitcast_convert_type(lax.shift_left(cv, 16),
                                              jnp.float32)
                bc = lax.bitcast_convert_type(jnp.bitwise_and(cv, _MASKHI),
                                              jnp.float32)
                slo = pl.ds(16 * t, 16)
                shi = pl.ds(DW + 16 * t, 16)
                hr[r, slo] = hr[r, slo] * ac
                hr[r, shi] = hr[r, shi] * bc

        plsc.parallel_loop(0, K, 1, unroll=4)(mrow)

    issue_idx(0, 0, 0)
    issue_idx(1, 1, 0)

    def body(g, carry):
        p = jnp.bitwise_and(g, 1)
        for b in (0, 1):
            src_v, dst_v, hr, cf, sg, sc_, ss, si = bufs[b]

            @pl.when(g >= 1)
            def _():
                pltpu.make_async_copy(hr, msg_sh.at[dst_v.at[1 - p]],
                                      ss).wait()

            wait_idx(b, p)
            chunk = wid + (2 * g + b) * NW
            pltpu.async_copy(h_hbm.at[src_v], hr, sg)
            pltpu.async_copy(coeff_hbm.at[pl.ds(chunk * K, K)], cf, sc_)
        for b in (0, 1):
            src_v, dst_v, hr, cf, sg, sc_, ss, si = bufs[b]
            chunk = wid + (2 * g + b) * NW
            pltpu.make_async_copy(h_hbm.at[src_v], hr, sg).wait()
            pltpu.make_async_copy(coeff_hbm.at[pl.ds(chunk * K, K)], cf,
                                  sc_).wait()

            @pl.when(2 * g + b + 2 < CPW)
            def _():
                issue_idx(b, 2 * g + b + 2, 1 - p)

            _mult(hr, cf)
            pltpu.async_copy(hr, msg_sh.at[dst_v.at[p]], ss, add=True)
        return carry

    lax.fori_loop(0, CPW // 2, body, 0)
    for b in (0, 1):
        src_v, dst_v, hr, cf, sg, sc_, ss, si = bufs[b]
        pltpu.make_async_copy(hr, msg_sh.at[dst_v.at[0]], ss).wait()

    tchunk = wid + (CPW - 1) * NW
    wait_idx(0, 0)
    pltpu.async_copy(h_hbm.at[src0_v], hr0_v, sg0).wait()
    pltpu.async_copy(coeff_hbm.at[pl.ds(tchunk * K, K)], cf0_v, sc0).wait()
    _mult(hr0_v, cf0_v)
    pltpu.async_copy(hr0_v, msg_sh.at[dst0_v.at[0]], ss0, add=True).wait()

    plsc.subcore_barrier()
    pltpu.sync_copy(msg_sh.at[pl.ds(base, RPT)], out_hbm.at[c, pl.ds(base, RPT)])


def _sc_scatter(h, coeff_p, src_p, dst_p):
    mesh = plsc.VectorSubcoreMesh(core_axis_name="c", subcore_axis_name="s",
                                  num_cores=NC, num_subcores=NS)
    fn = pl.kernel(
        _sc_body,
        out_type=jax.ShapeDtypeStruct((NC, NP, D), jnp.float32),
        mesh=mesh,
        scratch_types=[
            pltpu.VMEM((K,), jnp.int32),
            pltpu.VMEM((K,), jnp.int32),
            pltpu.VMEM((2, K), jnp.int32),
            pltpu.VMEM((2, K), jnp.int32),
            pltpu.VMEM((K, D), jnp.float32),
            pltpu.VMEM((K, D), jnp.float32),
            pltpu.VMEM((K, DW), jnp.int32),
            pltpu.VMEM((K, DW), jnp.int32),
            pltpu.VMEM_SHARED((NP, D), jnp.float32),
            pltpu.SemaphoreType.DMA,
            pltpu.SemaphoreType.DMA,
            pltpu.SemaphoreType.DMA,
            pltpu.SemaphoreType.DMA,
            pltpu.SemaphoreType.DMA,
            pltpu.SemaphoreType.DMA,
            pltpu.SemaphoreType.DMA,
            pltpu.SemaphoreType.DMA,
        ],
    )
    return fn(h, coeff_p, src_p, dst_p)


def _out_body(part_ref, attrs_ref, wlin_ref, wskt_ref, out_ref):
    m = part_ref[0] + part_ref[1]
    m2 = jnp.dot(m, wlin_ref[...], preferred_element_type=jnp.float32) * (
        1.0 / (math.sqrt(D) * AVG_NEIGH))
    attrs = attrs_ref[...]
    acc = attrs[:, 0][:, None] * jnp.dot(m2, wskt_ref[:, 0, :],
                                         preferred_element_type=jnp.float32)
    for j in range(1, NUM_ELEM):
        acc = acc + attrs[:, j][:, None] * jnp.dot(
            m2, wskt_ref[:, j, :], preferred_element_type=jnp.float32)
    out_ref[...] = acc * (1.0 / math.sqrt(D * NUM_ELEM))


def _final(partials, node_attrs, W_lin, W_skip_t):
    BN = 2000
    return pl.pallas_call(
        _out_body,
        out_shape=jax.ShapeDtypeStruct((N, D), jnp.float32),
        grid=(N // BN,),
        in_specs=[pl.BlockSpec((NC, BN, D), lambda i: (0, i, 0)),
                  pl.BlockSpec((BN, NUM_ELEM), lambda i: (i, 0)),
                  pl.BlockSpec((D, D), lambda i: (0, 0)),
                  pl.BlockSpec((D, NUM_ELEM, D), lambda i: (0, 0, 0))],
        out_specs=pl.BlockSpec((BN, D), lambda i: (i, 0)),
    )(partials, node_attrs, W_lin, W_skip_t)


def kernel(node_feats, node_attrs, edge_feats, edge_attrs, edge_index,
           W_up, W_r0, W_r1, W_r2, W_r3, W_lin, W_skip):
    src_p, dst_p = _split_idx(edge_index)
    h = _node_linear(node_feats, W_up)
    coeff = _edge_coeff(edge_feats, edge_attrs, W_r0, W_r1, W_r2, W_r3)
    partials = _sc_scatter(h, coeff, src_p, dst_p)
    return _final(partials, node_attrs, W_lin, W_skip)


# speedup vs baseline: 1.3461x; 1.1353x over previous
import functools
import math

import numpy as np
import jax
import jax.numpy as jnp
from jax import lax
from jax.experimental import pallas as pl
from jax.experimental.pallas import tpu as pltpu
from jax.experimental.pallas import tpu_sc as plsc

N = 10000
E = 320000
D = 128
NUM_ELEM = 10
NUM_BESSEL = 8
HIDDEN = 64
AVG_NEIGH = 32.0
_SILU_NORM = 1.6790532

NC = 2
NS = 16
L = 16
NW = NC * NS

K = 80
NP = 10240
RPT = NP // NS

BEB = 2560
EH0 = 63 * BEB
EH1 = 62 * BEB
_MASKHI = -65536

DW = D // 2


def _silu(x):
    return x * jax.nn.sigmoid(x) * _SILU_NORM


def _pack_rows(x):
    bits = lax.bitcast_convert_type(x, jnp.int32) + 0x8000
    return jnp.bitwise_or(lax.shift_right_logical(bits[:, :DW], 16),
                          jnp.bitwise_and(bits[:, DW:], _MASKHI))


def _split_body(ei_ref, src_ref, dst_ref):
    ei = ei_ref[...]
    src_ref[...] = ei[0]
    dst_ref[...] = ei[1]


def _split_idx(edge_index):
    return pl.pallas_call(
        _split_body,
        out_shape=(jax.ShapeDtypeStruct((E,), jnp.int32),
                   jax.ShapeDtypeStruct((E,), jnp.int32)),
    )(edge_index)


def _h_body(nf_ref, wup_ref, h_ref):
    h_ref[...] = jnp.dot(nf_ref[...], wup_ref[...],
                         preferred_element_type=jnp.float32) * (1.0 / math.sqrt(D))


def _node_linear(node_feats, W_up):
    BN = 2000
    return pl.pallas_call(
        _h_body,
        out_shape=jax.ShapeDtypeStruct((N, D), jnp.float32),
        grid=(N // BN,),
        in_specs=[pl.BlockSpec((BN, D), lambda i: (i, 0)),
                  pl.BlockSpec((D, D), lambda i: (0, 0))],
        out_specs=pl.BlockSpec((BN, D), lambda i: (i, 0)),
    )(node_feats, W_up)


def _coeff_body(ef_ref, ea_ref, w0_ref, w1_ref, w2_ref, w3_ref, out_ref):
    x = jnp.dot(ef_ref[...], w0_ref[...],
                preferred_element_type=jnp.float32) * (1.0 / math.sqrt(NUM_BESSEL))
    x = _silu(x)
    x = jnp.dot(x, w1_ref[...],
                preferred_element_type=jnp.float32) * (1.0 / math.sqrt(HIDDEN))
    x = _silu(x)
    x = jnp.dot(x, w2_ref[...],
                preferred_element_type=jnp.float32) * (1.0 / math.sqrt(HIDDEN))
    x = _silu(x)
    tw = jnp.dot(x, w3_ref[...],
                 preferred_element_type=jnp.float32) * (1.0 / math.sqrt(HIDDEN))
    out_ref[...] = _pack_rows(tw * ea_ref[...])


def _edge_coeff(edge_feats, edge_attrs, W_r0, W_r1, W_r2, W_r3p):
    ne = edge_feats.shape[0]
    return pl.pallas_call(
        _coeff_body,
        out_shape=jax.ShapeDtypeStruct((ne, DW), jnp.int32),
        grid=(ne // BEB,),
        in_specs=[pl.BlockSpec((BEB, NUM_BESSEL), lambda i: (i, 0)),
                  pl.BlockSpec((BEB, 1), lambda i: (i, 0)),
                  pl.BlockSpec((NUM_BESSEL, HIDDEN), lambda i: (0, 0)),
                  pl.BlockSpec((HIDDEN, HIDDEN), lambda i: (0, 0)),
                  pl.BlockSpec((HIDDEN, HIDDEN), lambda i: (0, 0)),
                  pl.BlockSpec((HIDDEN, D), lambda i: (0, 0))],
        out_specs=pl.BlockSpec((BEB, DW), lambda i: (i, 0)),
    )(edge_feats, edge_attrs, W_r0, W_r1, W_r2, W_r3p)


def _make_sc_body(ebase, cpw):
    pairs = cpw // 2
    has_tail = cpw % 2 == 1
    tail_row = 1 - ((pairs - 1) & 1)

    def _sc_body(h_hbm, coeff_hbm, src_hbm, dst_hbm, out_hbm,
                 src0_v, src1_v, dst0_v, dst1_v, hr0_v, hr1_v,
                 cf0_v, cf1_v, msg_sh,
                 sg0, sg1, sc0, sc1, ss0, ss1, si0, si1):
        c = lax.axis_index("c")
        s = lax.axis_index("s")
        wid = s * NC + c
        bufs = ((src0_v, dst0_v, hr0_v, cf0_v, sg0, sc0, ss0, si0),
                (src1_v, dst1_v, hr1_v, cf1_v, sg1, sc1, ss1, si1))

        def issue_idx(b, i, p):
            src_v, dst_v, _, _, _, _, _, si = bufs[b]
            chunk = wid + i * NW
            pltpu.async_copy(src_hbm.at[pl.ds(ebase + chunk * K, K)],
                             src_v, si)
            pltpu.async_copy(dst_hbm.at[pl.ds(ebase + chunk * K, K)],
                             dst_v.at[p], si)

        def wait_idx(b, p):
            src_v, dst_v, _, _, _, _, _, si = bufs[b]
            pltpu.make_async_copy(src_hbm.at[pl.ds(0, K)], src_v, si).wait()
            pltpu.make_async_copy(dst_hbm.at[pl.ds(0, K)], dst_v.at[p],
                                  si).wait()

        zero = jnp.zeros((L,), jnp.float32)

        def zrow(r, carry):
            for j in range(D // L):
                hr0_v[r, pl.ds(j * L, L)] = zero
            return carry

        lax.fori_loop(0, K, zrow, 0)
        base = s * RPT
        for t in range(RPT // K):
            pltpu.sync_copy(hr0_v, msg_sh.at[pl.ds(base + t * K, K)])
        plsc.subcore_barrier()

        def _mult(hr, cf):
            def mrow(r):
                for t in range(4):
                    cv = cf[r, pl.ds(16 * t, 16)]
                    ac = lax.bitcast_convert_type(lax.shift_left(cv, 16),
                                                  jnp.float32)
                    bc = lax.bitcast_convert_type(
                        jnp.bitwise_and(cv, _MASKHI), jnp.float32)
                    slo = pl.ds(16 * t, 16)
                    shi = pl.ds(DW + 16 * t, 16)
                    hr[r, slo] = hr[r, slo] * ac
                    hr[r, shi] = hr[r, shi] * bc

            plsc.parallel_loop(0, K, 1, unroll=4)(mrow)

        issue_idx(0, 0, 0)
        issue_idx(1, 1, 0)

        def body(g, carry):
            p = jnp.bitwise_and(g, 1)
            for b in (0, 1):
                src_v, dst_v, hr, cf, sg, sc_, ss, si = bufs[b]

                @pl.when(g >= 1)
                def _():
                    pltpu.make_async_copy(hr, msg_sh.at[dst_v.at[1 - p]],
                                          ss).wait()

                wait_idx(b, p)
                chunk = wid + (2 * g + b) * NW
                pltpu.async_copy(h_hbm.at[src_v], hr, sg)
                pltpu.async_copy(coeff_hbm.at[pl.ds(chunk * K, K)], cf, sc_)
            for b in (0, 1):
                src_v, dst_v, hr, cf, sg, sc_, ss, si = bufs[b]
                chunk = wid + (2 * g + b) * NW
                pltpu.make_async_copy(h_hbm.at[src_v], hr, sg).wait()
                pltpu.make_async_copy(coeff_hbm.at[pl.ds(chunk * K, K)], cf,
                                      sc_).wait()

                @pl.when(2 * g + b + 2 < cpw)
                def _():
                    issue_idx(b, 2 * g + b + 2, 1 - p)

                _mult(hr, cf)
                pltpu.async_copy(hr, msg_sh.at[dst_v.at[p]], ss, add=True)
            return carry

        lax.fori_loop(0, pairs, body, 0)
        for b in (0, 1):
            src_v, dst_v, hr, cf, sg, sc_, ss, si = bufs[b]
            pltpu.make_async_copy(hr, msg_sh.at[dst_v.at[0]], ss).wait()

        if has_tail:
            tchunk = wid + (cpw - 1) * NW
            wait_idx(0, tail_row)
            pltpu.async_copy(h_hbm.at[src0_v], hr0_v, sg0).wait()
            pltpu.async_copy(coeff_hbm.at[pl.ds(tchunk * K, K)], cf0_v,
                             sc0).wait()
            _mult(hr0_v, cf0_v)
            pltpu.async_copy(hr0_v, msg_sh.at[dst0_v.at[tail_row]], ss0,
                             add=True).wait()

        plsc.subcore_barrier()
        pltpu.sync_copy(msg_sh.at[pl.ds(base, RPT)],
                        out_hbm.at[c, pl.ds(base, RPT)])

    return _sc_body


def _sc_scatter(h, coeff_p, src_p, dst_p, ebase, cpw):
    mesh = plsc.VectorSubcoreMesh(core_axis_name="c", subcore_axis_name="s",
                                  num_cores=NC, num_subcores=NS)
    fn = pl.kernel(
        _make_sc_body(ebase, cpw),
        out_type=jax.ShapeDtypeStruct((NC, NP, D), jnp.float32),
        mesh=mesh,
        scratch_types=[
            pltpu.VMEM((K,), jnp.int32),
            pltpu.VMEM((K,), jnp.int32),
            pltpu.VMEM((2, K), jnp.int32),
            pltpu.VMEM((2, K), jnp.int32),
            pltpu.VMEM((K, D), jnp.float32),
            pltpu.VMEM((K, D), jnp.float32),
            pltpu.VMEM((K, DW), jnp.int32),
            pltpu.VMEM((K, DW), jnp.int32),
            pltpu.VMEM_SHARED((NP, D), jnp.float32),
            pltpu.SemaphoreType.DMA,
            pltpu.SemaphoreType.DMA,
            pltpu.SemaphoreType.DMA,
            pltpu.SemaphoreType.DMA,
            pltpu.SemaphoreType.DMA,
            pltpu.SemaphoreType.DMA,
            pltpu.SemaphoreType.DMA,
            pltpu.SemaphoreType.DMA,
        ],
    )
    return fn(h, coeff_p, src_p, dst_p)


def _out_body(part_ref, part2_ref, attrs_ref, wlin_ref, wskt_ref, out_ref):
    m = (part_ref[0] + part_ref[1]) + (part2_ref[0] + part2_ref[1])
    m2 = jnp.dot(m, wlin_ref[...], preferred_element_type=jnp.float32) * (
        1.0 / (math.sqrt(D) * AVG_NEIGH))
    attrs = attrs_ref[...]
    acc = attrs[:, 0][:, None] * jnp.dot(m2, wskt_ref[:, 0, :],
                                         preferred_element_type=jnp.float32)
    for j in range(1, NUM_ELEM):
        acc = acc + attrs[:, j][:, None] * jnp.dot(
            m2, wskt_ref[:, j, :], preferred_element_type=jnp.float32)
    out_ref[...] = acc * (1.0 / math.sqrt(D * NUM_ELEM))


def _final(partials, partials2, node_attrs, W_lin, W_skip_t):
    BN = 2000
    return pl.pallas_call(
        _out_body,
        out_shape=jax.ShapeDtypeStruct((N, D), jnp.float32),
        grid=(N // BN,),
        in_specs=[pl.BlockSpec((NC, BN, D), lambda i: (0, i, 0)),
                  pl.BlockSpec((NC, BN, D), lambda i: (0, i, 0)),
                  pl.BlockSpec((BN, NUM_ELEM), lambda i: (i, 0)),
                  pl.BlockSpec((D, D), lambda i: (0, 0)),
                  pl.BlockSpec((D, NUM_ELEM, D), lambda i: (0, 0, 0))],
        out_specs=pl.BlockSpec((BN, D), lambda i: (i, 0)),
    )(partials, partials2, node_attrs, W_lin, W_skip_t)


def kernel(node_feats, node_attrs, edge_feats, edge_attrs, edge_index,
           W_up, W_r0, W_r1, W_r2, W_r3, W_lin, W_skip):
    src_p, dst_p = _split_idx(edge_index)
    h = _node_linear(node_feats, W_up)
    coeff0 = _edge_coeff(edge_feats[:EH0], edge_attrs[:EH0],
                         W_r0, W_r1, W_r2, W_r3)
    partials0 = _sc_scatter(h, coeff0, src_p, dst_p, 0, EH0 // K // NW)
    coeff1 = _edge_coeff(edge_feats[EH0:], edge_attrs[EH0:],
                         W_r0, W_r1, W_r2, W_r3)
    partials1 = _sc_scatter(h, coeff1, src_p, dst_p, EH0, EH1 // K // NW)
    return _final(partials0, partials1, node_attrs, W_lin, W_skip)


# speedup vs baseline: 1.3684x; 1.0166x over previous
import functools
import math

import numpy as np
import jax
import jax.numpy as jnp
from jax import lax
from jax.experimental import pallas as pl
from jax.experimental.pallas import tpu as pltpu
from jax.experimental.pallas import tpu_sc as plsc

N = 10000
E = 320000
D = 128
NUM_ELEM = 10
NUM_BESSEL = 8
HIDDEN = 64
AVG_NEIGH = 32.0
_SILU_NORM = 1.6790532

NC = 2
NS = 16
L = 16
NW = NC * NS

K = 80
NP = 10240
RPT = NP // NS

BEB = 2560
SLICES = (32 * BEB, 31 * BEB, 31 * BEB, 31 * BEB)
_MASKHI = -65536

DW = D // 2


def _silu(x):
    return x * jax.nn.sigmoid(x) * _SILU_NORM


def _pack_rows(x):
    bits = lax.bitcast_convert_type(x, jnp.int32) + 0x8000
    return jnp.bitwise_or(lax.shift_right_logical(bits[:, :DW], 16),
                          jnp.bitwise_and(bits[:, DW:], _MASKHI))


def _split_body(ei_ref, src_ref, dst_ref):
    ei = ei_ref[...]
    src_ref[...] = ei[0]
    dst_ref[...] = ei[1]


def _split_idx(edge_index):
    return pl.pallas_call(
        _split_body,
        out_shape=(jax.ShapeDtypeStruct((E,), jnp.int32),
                   jax.ShapeDtypeStruct((E,), jnp.int32)),
    )(edge_index)


def _h_body(nf_ref, wup_ref, h_ref):
    h_ref[...] = jnp.dot(nf_ref[...], wup_ref[...],
                         preferred_element_type=jnp.float32) * (1.0 / math.sqrt(D))


def _node_linear(node_feats, W_up):
    BN = 2000
    return pl.pallas_call(
        _h_body,
        out_shape=jax.ShapeDtypeStruct((N, D), jnp.float32),
        grid=(N // BN,),
        in_specs=[pl.BlockSpec((BN, D), lambda i: (i, 0)),
                  pl.BlockSpec((D, D), lambda i: (0, 0))],
        out_specs=pl.BlockSpec((BN, D), lambda i: (i, 0)),
    )(node_feats, W_up)


def _coeff_body(ef_ref, ea_ref, w0_ref, w1_ref, w2_ref, w3_ref, out_ref):
    x = jnp.dot(ef_ref[...], w0_ref[...],
                preferred_element_type=jnp.float32) * (1.0 / math.sqrt(NUM_BESSEL))
    x = _silu(x)
    x = jnp.dot(x, w1_ref[...],
                preferred_element_type=jnp.float32) * (1.0 / math.sqrt(HIDDEN))
    x = _silu(x)
    x = jnp.dot(x, w2_ref[...],
                preferred_element_type=jnp.float32) * (1.0 / math.sqrt(HIDDEN))
    x = _silu(x)
    tw = jnp.dot(x, w3_ref[...],
                 preferred_element_type=jnp.float32) * (1.0 / math.sqrt(HIDDEN))
    out_ref[...] = _pack_rows(tw * ea_ref[...])


def _edge_coeff(edge_feats, edge_attrs, W_r0, W_r1, W_r2, W_r3p):
    ne = edge_feats.shape[0]
    return pl.pallas_call(
        _coeff_body,
        out_shape=jax.ShapeDtypeStruct((ne, DW), jnp.int32),
        grid=(ne // BEB,),
        in_specs=[pl.BlockSpec((BEB, NUM_BESSEL), lambda i: (i, 0)),
                  pl.BlockSpec((BEB, 1), lambda i: (i, 0)),
                  pl.BlockSpec((NUM_BESSEL, HIDDEN), lambda i: (0, 0)),
                  pl.BlockSpec((HIDDEN, HIDDEN), lambda i: (0, 0)),
                  pl.BlockSpec((HIDDEN, HIDDEN), lambda i: (0, 0)),
                  pl.BlockSpec((HIDDEN, D), lambda i: (0, 0))],
        out_specs=pl.BlockSpec((BEB, DW), lambda i: (i, 0)),
    )(edge_feats, edge_attrs, W_r0, W_r1, W_r2, W_r3p)


def _make_sc_body(ebase, cpw):
    pairs = cpw // 2
    has_tail = cpw % 2 == 1
    tail_row = 1 - ((pairs - 1) & 1)

    def _sc_body(h_hbm, coeff_hbm, src_hbm, dst_hbm, out_hbm,
                 src0_v, src1_v, dst0_v, dst1_v, hr0_v, hr1_v,
                 cf0_v, cf1_v, msg_sh,
                 sg0, sg1, sc0, sc1, ss0, ss1, si0, si1):
        c = lax.axis_index("c")
        s = lax.axis_index("s")
        wid = s * NC + c
        bufs = ((src0_v, dst0_v, hr0_v, cf0_v, sg0, sc0, ss0, si0),
                (src1_v, dst1_v, hr1_v, cf1_v, sg1, sc1, ss1, si1))

        def issue_idx(b, i, p):
            src_v, dst_v, _, _, _, _, _, si = bufs[b]
            chunk = wid + i * NW
            pltpu.async_copy(src_hbm.at[pl.ds(ebase + chunk * K, K)],
                             src_v, si)
            pltpu.async_copy(dst_hbm.at[pl.ds(ebase + chunk * K, K)],
                             dst_v.at[p], si)

        def wait_idx(b, p):
            src_v, dst_v, _, _, _, _, _, si = bufs[b]
            pltpu.make_async_copy(src_hbm.at[pl.ds(0, K)], src_v, si).wait()
            pltpu.make_async_copy(dst_hbm.at[pl.ds(0, K)], dst_v.at[p],
                                  si).wait()

        zero = jnp.zeros((L,), jnp.float32)

        def zrow(r, carry):
            for j in range(D // L):
                hr0_v[r, pl.ds(j * L, L)] = zero
            return carry

        lax.fori_loop(0, K, zrow, 0)
        base = s * RPT
        for t in range(RPT // K):
            pltpu.sync_copy(hr0_v, msg_sh.at[pl.ds(base + t * K, K)])
        plsc.subcore_barrier()

        def _mult(hr, cf):
            def mrow(r):
                for t in range(4):
                    cv = cf[r, pl.ds(16 * t, 16)]
                    ac = lax.bitcast_convert_type(lax.shift_left(cv, 16),
                                                  jnp.float32)
                    bc = lax.bitcast_convert_type(
                        jnp.bitwise_and(cv, _MASKHI), jnp.float32)
                    slo = pl.ds(16 * t, 16)
                    shi = pl.ds(DW + 16 * t, 16)
                    hr[r, slo] = hr[r, slo] * ac
                    hr[r, shi] = hr[r, shi] * bc

            plsc.parallel_loop(0, K, 1, unroll=4)(mrow)

        issue_idx(0, 0, 0)
        issue_idx(1, 1, 0)

        def body(g, carry):
            p = jnp.bitwise_and(g, 1)
            for b in (0, 1):
                src_v, dst_v, hr, cf, sg, sc_, ss, si = bufs[b]

                @pl.when(g >= 1)
                def _():
                    pltpu.make_async_copy(hr, msg_sh.at[dst_v.at[1 - p]],
                                          ss).wait()

                wait_idx(b, p)
                chunk = wid + (2 * g + b) * NW
                pltpu.async_copy(h_hbm.at[src_v], hr, sg)
                pltpu.async_copy(coeff_hbm.at[pl.ds(chunk * K, K)], cf, sc_)
            for b in (0, 1):
                src_v, dst_v, hr, cf, sg, sc_, ss, si = bufs[b]
                chunk = wid + (2 * g + b) * NW
                pltpu.make_async_copy(h_hbm.at[src_v], hr, sg).wait()
                pltpu.make_async_copy(coeff_hbm.at[pl.ds(chunk * K, K)], cf,
                                      sc_).wait()

                @pl.when(2 * g + b + 2 < cpw)
                def _():
                    issue_idx(b, 2 * g + b + 2, 1 - p)

                _mult(hr, cf)
                pltpu.async_copy(hr, msg_sh.at[dst_v.at[p]], ss, add=True)
            return carry

        lax.fori_loop(0, pairs, body, 0)
        for b in (0, 1):
            src_v, dst_v, hr, cf, sg, sc_, ss, si = bufs[b]
            pltpu.make_async_copy(hr, msg_sh.at[dst_v.at[0]], ss).wait()

        if has_tail:
            tchunk = wid + (cpw - 1) * NW
            wait_idx(0, tail_row)
            pltpu.async_copy(h_hbm.at[src0_v], hr0_v, sg0).wait()
            pltpu.async_copy(coeff_hbm.at[pl.ds(tchunk * K, K)], cf0_v,
                             sc0).wait()
            _mult(hr0_v, cf0_v)
            pltpu.async_copy(hr0_v, msg_sh.at[dst0_v.at[tail_row]], ss0,
                             add=True).wait()

        plsc.subcore_barrier()
        pltpu.sync_copy(msg_sh.at[pl.ds(base, RPT)],
                        out_hbm.at[c, pl.ds(base, RPT)])

    return _sc_body


def _sc_scatter(h, coeff_p, src_p, dst_p, ebase, cpw):
    mesh = plsc.VectorSubcoreMesh(core_axis_name="c", subcore_axis_name="s",
                                  num_cores=NC, num_subcores=NS)
    fn = pl.kernel(
        _make_sc_body(ebase, cpw),
        out_type=jax.ShapeDtypeStruct((NC, NP, D), jnp.float32),
        mesh=mesh,
        scratch_types=[
            pltpu.VMEM((K,), jnp.int32),
            pltpu.VMEM((K,), jnp.int32),
            pltpu.VMEM((2, K), jnp.int32),
            pltpu.VMEM((2, K), jnp.int32),
            pltpu.VMEM((K, D), jnp.float32),
            pltpu.VMEM((K, D), jnp.float32),
            pltpu.VMEM((K, DW), jnp.int32),
            pltpu.VMEM((K, DW), jnp.int32),
            pltpu.VMEM_SHARED((NP, D), jnp.float32),
            pltpu.SemaphoreType.DMA,
            pltpu.SemaphoreType.DMA,
            pltpu.SemaphoreType.DMA,
            pltpu.SemaphoreType.DMA,
            pltpu.SemaphoreType.DMA,
            pltpu.SemaphoreType.DMA,
            pltpu.SemaphoreType.DMA,
            pltpu.SemaphoreType.DMA,
        ],
    )
    return fn(h, coeff_p, src_p, dst_p)


def _out_body(p0_ref, p1_ref, p2_ref, p3_ref, attrs_ref, wlin_ref, wskt_ref,
              out_ref):
    m = ((p0_ref[0] + p0_ref[1]) + (p1_ref[0] + p1_ref[1])
         + (p2_ref[0] + p2_ref[1]) + (p3_ref[0] + p3_ref[1]))
    m2 = jnp.dot(m, wlin_ref[...], preferred_element_type=jnp.float32) * (
        1.0 / (math.sqrt(D) * AVG_NEIGH))
    attrs = attrs_ref[...]
    acc = attrs[:, 0][:, None] * jnp.dot(m2, wskt_ref[:, 0, :],
                                         preferred_element_type=jnp.float32)
    for j in range(1, NUM_ELEM):
        acc = acc + attrs[:, j][:, None] * jnp.dot(
            m2, wskt_ref[:, j, :], preferred_element_type=jnp.float32)
    out_ref[...] = acc * (1.0 / math.sqrt(D * NUM_ELEM))


def _final(parts, node_attrs, W_lin, W_skip_t):
    BN = 2000
    return pl.pallas_call(
        _out_body,
        out_shape=jax.ShapeDtypeStruct((N, D), jnp.float32),
        grid=(N // BN,),
        in_specs=[pl.BlockSpec((NC, BN, D), lambda i: (0, i, 0))] * 4 +
                 [pl.BlockSpec((BN, NUM_ELEM), lambda i: (i, 0)),
                  pl.BlockSpec((D, D), lambda i: (0, 0)),
                  pl.BlockSpec((D, NUM_ELEM, D), lambda i: (0, 0, 0))],
        out_specs=pl.BlockSpec((BN, D), lambda i: (i, 0)),
    )(*parts, node_attrs, W_lin, W_skip_t)


def kernel(node_feats, node_attrs, edge_feats, edge_attrs, edge_index,
           W_up, W_r0, W_r1, W_r2, W_r3, W_lin, W_skip):
    src_p, dst_p = _split_idx(edge_index)
    h = _node_linear(node_feats, W_up)
    parts = []
    eb = 0
    for ne in SLICES:
        coeff_q = _edge_coeff(edge_feats[eb:eb + ne], edge_attrs[eb:eb + ne],
                              W_r0, W_r1, W_r2, W_r3)
        parts.append(_sc_scatter(h, coeff_q, src_p, dst_p, eb,
                                 ne // K // NW))
        eb += ne
    return _final(parts, node_attrs, W_lin, W_skip)


# speedup vs baseline: 1.3691x; 1.0005x over previous
import functools
import math

import numpy as np
import jax
import jax.numpy as jnp
from jax import lax
from jax.experimental import pallas as pl
from jax.experimental.pallas import tpu as pltpu
from jax.experimental.pallas import tpu_sc as plsc

N = 10000
E = 320000
D = 128
NUM_ELEM = 10
NUM_BESSEL = 8
HIDDEN = 64
AVG_NEIGH = 32.0
_SILU_NORM = 1.6790532

NC = 2
NS = 16
L = 16
NW = NC * NS

K = 80
NP = 10240
RPT = NP // NS

BEB = 2560
SLICES = (32 * BEB, 31 * BEB, 31 * BEB, 31 * BEB)
_MASKHI = -65536

DW = D // 2


def _silu(x):
    return x * jax.nn.sigmoid(x) * _SILU_NORM


def _pack_rows(x):
    bits = lax.bitcast_convert_type(x, jnp.int32) + 0x8000
    return jnp.bitwise_or(lax.shift_right_logical(bits[:, :DW], 16),
                          jnp.bitwise_and(bits[:, DW:], _MASKHI))


def _split_body(ei_ref, src_ref, dst_ref):
    ei = ei_ref[...]
    src_ref[...] = ei[0]
    dst_ref[...] = ei[1]


def _split_idx(edge_index):
    return pl.pallas_call(
        _split_body,
        out_shape=(jax.ShapeDtypeStruct((E,), jnp.int32),
                   jax.ShapeDtypeStruct((E,), jnp.int32)),
    )(edge_index)


def _h_body(nf_ref, wup_ref, h_ref):
    h_ref[...] = jnp.dot(nf_ref[...], wup_ref[...],
                         preferred_element_type=jnp.float32) * (1.0 / math.sqrt(D))


def _node_linear(node_feats, W_up):
    BN = 2000
    return pl.pallas_call(
        _h_body,
        out_shape=jax.ShapeDtypeStruct((N, D), jnp.float32),
        grid=(N // BN,),
        in_specs=[pl.BlockSpec((BN, D), lambda i: (i, 0)),
                  pl.BlockSpec((D, D), lambda i: (0, 0))],
        out_specs=pl.BlockSpec((BN, D), lambda i: (i, 0)),
    )(node_feats, W_up)


def _coeff_body(ef_ref, ea_ref, w0_ref, w1_ref, w2_ref, w3_ref, out_ref):
    ea = ea_ref[...]
    x = jnp.dot(ef_ref[...], w0_ref[...],
                preferred_element_type=jnp.float32) * (1.0 / math.sqrt(NUM_BESSEL))
    x = _silu(x)
    x = jnp.dot(x, w1_ref[...],
                preferred_element_type=jnp.float32) * (1.0 / math.sqrt(HIDDEN))
    x = _silu(x)
    x = jnp.dot(x, w2_ref[...],
                preferred_element_type=jnp.float32) * (1.0 / math.sqrt(HIDDEN))
    x = _silu(x)
    tw = jnp.dot(x, w3_ref[...],
                 preferred_element_type=jnp.float32) * (1.0 / math.sqrt(HIDDEN))
    out_ref[...] = _pack_rows(tw * ea)


def _edge_coeff(edge_feats, edge_attrs, W_r0, W_r1, W_r2, W_r3p):
    ne = edge_feats.shape[0]
    return pl.pallas_call(
        _coeff_body,
        out_shape=jax.ShapeDtypeStruct((ne, DW), jnp.int32),
        grid=(ne // BEB,),
        in_specs=[pl.BlockSpec((BEB, NUM_BESSEL), lambda i: (i, 0)),
                  pl.BlockSpec((BEB, 1), lambda i: (i, 0)),
                  pl.BlockSpec((NUM_BESSEL, HIDDEN), lambda i: (0, 0)),
                  pl.BlockSpec((HIDDEN, HIDDEN), lambda i: (0, 0)),
                  pl.BlockSpec((HIDDEN, HIDDEN), lambda i: (0, 0)),
                  pl.BlockSpec((HIDDEN, D), lambda i: (0, 0))],
        out_specs=pl.BlockSpec((BEB, DW), lambda i: (i, 0)),
    )(edge_feats, edge_attrs, W_r0, W_r1, W_r2, W_r3p)


def _make_sc_body(ebase, cpw):
    pairs = cpw // 2
    has_tail = cpw % 2 == 1
    tail_row = 1 - ((pairs - 1) & 1)

    def _sc_body(h_hbm, coeff_hbm, src_hbm, dst_hbm, out_hbm,
                 src0_v, src1_v, dst0_v, dst1_v, hr0_v, hr1_v,
                 cf0_v, cf1_v, msg_sh,
                 sg0, sg1, sc0, sc1, ss0, ss1, si0, si1):
        c = lax.axis_index("c")
        s = lax.axis_index("s")
        wid = s * NC + c
        bufs = ((src0_v, dst0_v, hr0_v, cf0_v, sg0, sc0, ss0, si0),
                (src1_v, dst1_v, hr1_v, cf1_v, sg1, sc1, ss1, si1))

        def issue_idx(b, i, p):
            src_v, dst_v, _, _, _, _, _, si = bufs[b]
            chunk = wid + i * NW
            pltpu.async_copy(src_hbm.at[pl.ds(ebase + chunk * K, K)],
                             src_v, si)
            pltpu.async_copy(dst_hbm.at[pl.ds(ebase + chunk * K, K)],
                             dst_v.at[p], si)

        def wait_idx(b, p):
            src_v, dst_v, _, _, _, _, _, si = bufs[b]
            pltpu.make_async_copy(src_hbm.at[pl.ds(0, K)], src_v, si).wait()
            pltpu.make_async_copy(dst_hbm.at[pl.ds(0, K)], dst_v.at[p],
                                  si).wait()

        zero = jnp.zeros((L,), jnp.float32)

        def zrow(r, carry):
            for j in range(D // L):
                hr0_v[r, pl.ds(j * L, L)] = zero
            return carry

        lax.fori_loop(0, K, zrow, 0)
        base = s * RPT
        for t in range(RPT // K):
            pltpu.sync_copy(hr0_v, msg_sh.at[pl.ds(base + t * K, K)])
        plsc.subcore_barrier()

        def _mult(hr, cf):
            def mrow(r):
                for t in range(4):
                    cv = cf[r, pl.ds(16 * t, 16)]
                    ac = lax.bitcast_convert_type(lax.shift_left(cv, 16),
                                                  jnp.float32)
                    bc = lax.bitcast_convert_type(
                        jnp.bitwise_and(cv, _MASKHI), jnp.float32)
                    slo = pl.ds(16 * t, 16)
                    shi = pl.ds(DW + 16 * t, 16)
                    hr[r, slo] = hr[r, slo] * ac
                    hr[r, shi] = hr[r, shi] * bc

            plsc.parallel_loop(0, K, 1, unroll=4)(mrow)

        issue_idx(0, 0, 0)
        issue_idx(1, 1, 0)

        def body(g, carry):
            p = jnp.bitwise_and(g, 1)
            for b in (0, 1):
                src_v, dst_v, hr, cf, sg, sc_, ss, si = bufs[b]

                @pl.when(g >= 1)
                def _():
                    pltpu.make_async_copy(hr, msg_sh.at[dst_v.at[1 - p]],
                                          ss).wait()

                wait_idx(b, p)
                chunk = wid + (2 * g + b) * NW
                pltpu.async_copy(h_hbm.at[src_v], hr, sg)
                pltpu.async_copy(coeff_hbm.at[pl.ds(chunk * K, K)], cf, sc_)
            for b in (0, 1):
                src_v, dst_v, hr, cf, sg, sc_, ss, si = bufs[b]
                chunk = wid + (2 * g + b) * NW
                pltpu.make_async_copy(h_hbm.at[src_v], hr, sg).wait()
                pltpu.make_async_copy(coeff_hbm.at[pl.ds(chunk * K, K)], cf,
                                      sc_).wait()

                @pl.when(2 * g + b + 2 < cpw)
                def _():
                    issue_idx(b, 2 * g + b + 2, 1 - p)

                _mult(hr, cf)
                pltpu.async_copy(hr, msg_sh.at[dst_v.at[p]], ss, add=True)
            return carry

        lax.fori_loop(0, pairs, body, 0)
        for b in (0, 1):
            src_v, dst_v, hr, cf, sg, sc_, ss, si = bufs[b]
            pltpu.make_async_copy(hr, msg_sh.at[dst_v.at[0]], ss).wait()

        if has_tail:
            tchunk = wid + (cpw - 1) * NW
            wait_idx(0, tail_row)
            pltpu.async_copy(h_hbm.at[src0_v], hr0_v, sg0).wait()
            pltpu.async_copy(coeff_hbm.at[pl.ds(tchunk * K, K)], cf0_v,
                             sc0).wait()
            _mult(hr0_v, cf0_v)
            pltpu.async_copy(hr0_v, msg_sh.at[dst0_v.at[tail_row]], ss0,
                             add=True).wait()

        plsc.subcore_barrier()
        pltpu.sync_copy(msg_sh.at[pl.ds(base, RPT)],
                        out_hbm.at[c, pl.ds(base, RPT)])

    return _sc_body


def _sc_scatter(h, coeff_p, src_p, dst_p, ebase, cpw):
    mesh = plsc.VectorSubcoreMesh(core_axis_name="c", subcore_axis_name="s",
                                  num_cores=NC, num_subcores=NS)
    fn = pl.kernel(
        _make_sc_body(ebase, cpw),
        out_type=jax.ShapeDtypeStruct((NC, NP, D), jnp.float32),
        mesh=mesh,
        scratch_types=[
            pltpu.VMEM((K,), jnp.int32),
            pltpu.VMEM((K,), jnp.int32),
            pltpu.VMEM((2, K), jnp.int32),
            pltpu.VMEM((2, K), jnp.int32),
            pltpu.VMEM((K, D), jnp.float32),
            pltpu.VMEM((K, D), jnp.float32),
            pltpu.VMEM((K, DW), jnp.int32),
            pltpu.VMEM((K, DW), jnp.int32),
            pltpu.VMEM_SHARED((NP, D), jnp.float32),
            pltpu.SemaphoreType.DMA,
            pltpu.SemaphoreType.DMA,
            pltpu.SemaphoreType.DMA,
            pltpu.SemaphoreType.DMA,
            pltpu.SemaphoreType.DMA,
            pltpu.SemaphoreType.DMA,
            pltpu.SemaphoreType.DMA,
            pltpu.SemaphoreType.DMA,
        ],
    )
    return fn(h, coeff_p, src_p, dst_p)


def _out_body(p0_ref, p1_ref, p2_ref, p3_ref, attrs_ref, wlin_ref, wskt_ref,
              out_ref):
    m = ((p0_ref[0] + p0_ref[1]) + (p1_ref[0] + p1_ref[1])
         + (p2_ref[0] + p2_ref[1]) + (p3_ref[0] + p3_ref[1]))
    m2 = jnp.dot(m, wlin_ref[...], preferred_element_type=jnp.float32) * (
        1.0 / (math.sqrt(D) * AVG_NEIGH))
    attrs = attrs_ref[...]
    acc = attrs[:, 0][:, None] * jnp.dot(m2, wskt_ref[:, 0, :],
                                         preferred_element_type=jnp.float32)
    for j in range(1, NUM_ELEM):
        acc = acc + attrs[:, j][:, None] * jnp.dot(
            m2, wskt_ref[:, j, :], preferred_element_type=jnp.float32)
    out_ref[...] = acc * (1.0 / math.sqrt(D * NUM_ELEM))


def _final(parts, node_attrs, W_lin, W_skip_t):
    BN = 2000
    return pl.pallas_call(
        _out_body,
        out_shape=jax.ShapeDtypeStruct((N, D), jnp.float32),
        grid=(N // BN,),
        in_specs=[pl.BlockSpec((NC, BN, D), lambda i: (0, i, 0))] * 4 +
                 [pl.BlockSpec((BN, NUM_ELEM), lambda i: (i, 0)),
                  pl.BlockSpec((D, D), lambda i: (0, 0)),
                  pl.BlockSpec((D, NUM_ELEM, D), lambda i: (0, 0, 0))],
        out_specs=pl.BlockSpec((BN, D), lambda i: (i, 0)),
    )(*parts, node_attrs, W_lin, W_skip_t)


def kernel(node_feats, node_attrs, edge_feats, edge_attrs, edge_index,
           W_up, W_r0, W_r1, W_r2, W_r3, W_lin, W_skip):
    src_p, dst_p = _split_idx(edge_index)
    h = _node_linear(node_feats, W_up)
    parts = []
    eb = 0
    for ne in SLICES:
        coeff_q = _edge_coeff(edge_feats[eb:eb + ne], edge_attrs[eb:eb + ne],
                              W_r0, W_r1, W_r2, W_r3)
        parts.append(_sc_scatter(h, coeff_q, src_p, dst_p, eb,
                                 ne // K // NW))
        eb += ne
    return _final(parts, node_attrs, W_lin, W_skip)
